# baseline (speedup 1.0000x reference)
import functools
import math

import jax
import jax.numpy as jnp
from jax import lax
from jax.experimental import pallas as pl
from jax.experimental.pallas import tpu as pltpu
from jax.experimental.pallas import tpu_sc as plsc

_NUM_BUCKETS = 32
_NUM_HEADS = 16
_Q = 4096
_K = 4096
_D = 8192
_BQ = 64

_SC_CORES = 2
_SC_SUBCORES = 16
_SC_WORKERS = _SC_CORES * _SC_SUBCORES
_IDX_PER_WORKER = _D // _SC_WORKERS


def _bucket_kernel(shift_ref, out_ref):
    shift = shift_ref[0]
    t = jax.lax.broadcasted_iota(jnp.int32, (1, _D), 1)
    rel = t - (_Q - 1) + shift
    n = -rel
    half = _NUM_BUCKETS // 2
    neg = jnp.where(n < 0, half, 0)
    n = jnp.abs(n)
    max_exact = half // 2
    scale = (half - max_exact) / math.log(128 / max_exact)
    log_val = (jnp.log(n.astype(jnp.float32) / max_exact + 1e-10)
               * scale).astype(jnp.int32)
    bucket = jnp.where(n < max_exact, n, max_exact + log_val)
    out_ref[...] = jnp.clip(bucket, 0, half - 1) + neg


def _sc_lookup_kernel(table_hbm, idx_hbm, out_hbm, table_v, idx_v, rows_v):
    wid = lax.axis_index("s") * _SC_CORES + lax.axis_index("c")
    base = wid * _IDX_PER_WORKER
    pltpu.sync_copy(table_hbm, table_v)
    pltpu.sync_copy(idx_hbm.at[pl.ds(base, _IDX_PER_WORKER)], idx_v)

    for g in range(_IDX_PER_WORKER // 16):
        idx16 = idx_v[pl.ds(g * 16, 16)]
        row_base = idx16 * _NUM_HEADS
        for h in range(_NUM_HEADS):
            vals = plsc.load_gather(table_v, [row_base + h])
            rows_v[h, pl.ds(g * 16, 16)] = vals
    pltpu.sync_copy(rows_v, out_hbm.at[:, pl.ds(base, _IDX_PER_WORKER)])


_BAND = 91
_PATCH = 384
_PWIN = 640


def _expand_kernel(shift_ref, diag_ref, out_ref, wb_ref):
    i = pl.program_id(0)
    shift = shift_ref[0]
    base_thr = i * _BQ - _BAND - shift
    lb = jnp.clip((base_thr >> 7) << 7, 0, _K - _PATCH)
    lb = pl.multiple_of(lb, 128)

    start = _Q - (i + 1) * _BQ
    pos = start + lb
    pos_al = pl.multiple_of(jnp.clip((pos >> 7) << 7, 0, _D - _PWIN), 128)
    awin = diag_ref[:, pl.ds(pos_al, _PWIN)]
    wb_ref[...] = pltpu.roll(awin, pos_al - pos + _PWIN, axis=1)

    c_lo = jnp.broadcast_to(diag_ref[:, 0:1], (_NUM_HEADS, _K))
    c_hi = jnp.broadcast_to(diag_ref[:, _D - 1:_D], (_NUM_HEADS, _K))
    lane = jax.lax.broadcasted_iota(jnp.int32, (_NUM_HEADS, _K), 1)
    fill = jnp.where(lane <= base_thr, c_lo, c_hi)
    out_ref[...] = jnp.broadcast_to(fill[:, None, :], (_NUM_HEADS, _BQ, _K))
    for r in range(_BQ):
        off = _BQ - 1 - r
        out_ref[:, r, pl.ds(lb, _PATCH)] = wb_ref[:, off:off + _PATCH]


def kernel(query_len, key_len, relative_attention_bias):
    shift = jnp.asarray(key_len - query_len, jnp.int32).reshape(1)

    buckets = pl.pallas_call(
        _bucket_kernel,
        grid_spec=pltpu.PrefetchScalarGridSpec(
            num_scalar_prefetch=1,
            grid=(1,),
            in_specs=[],
            out_specs=pl.BlockSpec((1, _D), lambda i, s: (0, 0)),
        ),
        out_shape=jax.ShapeDtypeStruct((1, _D), jnp.int32),
    )(shift).reshape(_D)

    lookup = pl.kernel(
        _sc_lookup_kernel,
        out_type=jax.ShapeDtypeStruct((_NUM_HEADS, _D), jnp.float32),
        mesh=plsc.VectorSubcoreMesh(core_axis_name="c", subcore_axis_name="s"),
        scratch_types=[
            pltpu.VMEM((_NUM_BUCKETS * _NUM_HEADS,), jnp.float32),
            pltpu.VMEM((_IDX_PER_WORKER,), jnp.int32),
            pltpu.VMEM((_NUM_HEADS, _IDX_PER_WORKER), jnp.float32),
        ],
        compiler_params=pltpu.CompilerParams(needs_layout_passes=False),
    )
    diag = lookup(relative_attention_bias.reshape(-1), buckets)

    out = pl.pallas_call(
        _expand_kernel,
        grid_spec=pltpu.PrefetchScalarGridSpec(
            num_scalar_prefetch=1,
            grid=(_Q // _BQ,),
            in_specs=[pl.BlockSpec((_NUM_HEADS, _D), lambda i, s: (0, 0))],
            out_specs=pl.BlockSpec((_NUM_HEADS, _BQ, _K), lambda i, s: (0, i, 0)),
            scratch_shapes=[pltpu.VMEM((_NUM_HEADS, _PWIN), jnp.float32)],
        ),
        out_shape=jax.ShapeDtypeStruct((_NUM_HEADS, _Q, _K), jnp.float32),
        compiler_params=pltpu.CompilerParams(
            dimension_semantics=("arbitrary",),
        ),
    )(shift, diag)
    return out[None]


# speedup vs baseline: 1.0032x; 1.0032x over previous
import math

import jax
import jax.numpy as jnp
from jax import lax
from jax.experimental import pallas as pl
from jax.experimental.pallas import tpu as pltpu
from jax.experimental.pallas import tpu_sc as plsc

_NUM_BUCKETS = 32
_NUM_HEADS = 16
_Q = 4096
_K = 4096
_D = 8192
_BQ = 64

_SC_CORES = 2
_SC_SUBCORES = 16
_SC_WORKERS = _SC_CORES * _SC_SUBCORES
_IDX_PER_WORKER = _D // _SC_WORKERS


def _bucket_kernel(shift_ref, out_ref):
    shift = shift_ref[0]
    t = jax.lax.broadcasted_iota(jnp.int32, (1, _D), 1)
    rel = t - (_Q - 1) + shift
    n = -rel
    half = _NUM_BUCKETS // 2
    neg = jnp.where(n < 0, half, 0)
    n = jnp.abs(n)
    max_exact = half // 2
    scale = (half - max_exact) / math.log(128 / max_exact)
    log_val = (jnp.log(n.astype(jnp.float32) / max_exact + 1e-10)
               * scale).astype(jnp.int32)
    bucket = jnp.where(n < max_exact, n, max_exact + log_val)
    out_ref[...] = jnp.clip(bucket, 0, half - 1) + neg


def _sc_lookup_kernel(table_hbm, idx_hbm, out_hbm, table_v, idx_v, rows_v):
    wid = lax.axis_index("s") * _SC_CORES + lax.axis_index("c")
    base = wid * _IDX_PER_WORKER
    pltpu.sync_copy(table_hbm, table_v)
    pltpu.sync_copy(idx_hbm.at[pl.ds(base, _IDX_PER_WORKER)], idx_v)

    for g in range(_IDX_PER_WORKER // 16):
        idx16 = idx_v[pl.ds(g * 16, 16)]
        row_base = idx16 * _NUM_HEADS
        for h in range(_NUM_HEADS):
            vals = plsc.load_gather(table_v, [row_base + h])
            rows_v[h, pl.ds(g * 16, 16)] = vals
    pltpu.sync_copy(rows_v, out_hbm.at[:, pl.ds(base, _IDX_PER_WORKER)])


_BAND = 91
_PATCH = 384
_PWIN = 640


def _expand_kernel(shift_ref, diag_ref, out_ref, wb_ref):
    i = pl.program_id(0)
    shift = shift_ref[0]
    base_thr = i * _BQ - _BAND - shift
    lb = jnp.clip((base_thr >> 7) << 7, 0, _K - _PATCH)
    lb = pl.multiple_of(lb, 128)

    start = _Q - (i + 1) * _BQ
    pos = start + lb
    pos_al = pl.multiple_of(jnp.clip((pos >> 7) << 7, 0, _D - _PWIN), 128)
    awin = diag_ref[:, pl.ds(pos_al, _PWIN)]
    wb_ref[...] = pltpu.roll(awin, pos_al - pos + _PWIN, axis=1)

    c_lo = jnp.broadcast_to(diag_ref[:, 0:1], (_NUM_HEADS, _K))
    c_hi = jnp.broadcast_to(diag_ref[:, _D - 1:_D], (_NUM_HEADS, _K))
    lane = jax.lax.broadcasted_iota(jnp.int32, (_NUM_HEADS, _K), 1)
    fill = jnp.where(lane <= base_thr, c_lo, c_hi)
    out_ref[...] = jnp.broadcast_to(fill[:, None, :], (_NUM_HEADS, _BQ, _K))
    for r in range(_BQ):
        off = _BQ - 1 - r
        out_ref[:, r, pl.ds(lb, _PATCH)] = wb_ref[:, off:off + _PATCH]


def kernel(query_len, key_len, relative_attention_bias):
    shift = jnp.asarray(key_len - query_len, jnp.int32).reshape(1)

    buckets = pl.pallas_call(
        _bucket_kernel,
        grid_spec=pltpu.PrefetchScalarGridSpec(
            num_scalar_prefetch=1,
            grid=(1,),
            in_specs=[],
            out_specs=pl.BlockSpec((1, _D), lambda i, s: (0, 0)),
        ),
        out_shape=jax.ShapeDtypeStruct((1, _D), jnp.int32),
    )(shift).reshape(_D)

    lookup = pl.kernel(
        _sc_lookup_kernel,
        out_type=jax.ShapeDtypeStruct((_NUM_HEADS, _D), jnp.float32),
        mesh=plsc.VectorSubcoreMesh(core_axis_name="c", subcore_axis_name="s"),
        scratch_types=[
            pltpu.VMEM((_NUM_BUCKETS * _NUM_HEADS,), jnp.float32),
            pltpu.VMEM((_IDX_PER_WORKER,), jnp.int32),
            pltpu.VMEM((_NUM_HEADS, _IDX_PER_WORKER), jnp.float32),
        ],
        compiler_params=pltpu.CompilerParams(needs_layout_passes=False),
    )
    diag = lookup(relative_attention_bias.reshape(-1), buckets)

    out = pl.pallas_call(
        _expand_kernel,
        grid_spec=pltpu.PrefetchScalarGridSpec(
            num_scalar_prefetch=1,
            grid=(_Q // _BQ,),
            in_specs=[pl.BlockSpec((_NUM_HEADS, _D), lambda i, s: (0, 0))],
            out_specs=pl.BlockSpec((_NUM_HEADS, _BQ, _K), lambda i, s: (0, i, 0)),
            scratch_shapes=[pltpu.VMEM((_NUM_HEADS, _PWIN), jnp.float32)],
        ),
        out_shape=jax.ShapeDtypeStruct((_NUM_HEADS, _Q, _K), jnp.float32),
        compiler_params=pltpu.CompilerParams(
            dimension_semantics=("arbitrary",),
        ),
    )(shift, diag)
    return out[None]


# speedup vs baseline: 1.0075x; 1.0042x over previous
import math

import jax
import jax.numpy as jnp
from jax import lax
from jax.experimental import pallas as pl
from jax.experimental.pallas import tpu as pltpu
from jax.experimental.pallas import tpu_sc as plsc

_NUM_BUCKETS = 32
_NUM_HEADS = 16
_Q = 4096
_K = 4096
_D = 8192
_BQ = 32

_SC_CORES = 2
_SC_SUBCORES = 16
_SC_WORKERS = _SC_CORES * _SC_SUBCORES
_IDX_PER_WORKER = _D // _SC_WORKERS


def _bucket_kernel(shift_ref, out_ref):
    shift = shift_ref[0]
    t = jax.lax.broadcasted_iota(jnp.int32, (1, _D), 1)
    rel = t - (_Q - 1) + shift
    n = -rel
    half = _NUM_BUCKETS // 2
    neg = jnp.where(n < 0, half, 0)
    n = jnp.abs(n)
    max_exact = half // 2
    scale = (half - max_exact) / math.log(128 / max_exact)
    log_val = (jnp.log(n.astype(jnp.float32) / max_exact + 1e-10)
               * scale).astype(jnp.int32)
    bucket = jnp.where(n < max_exact, n, max_exact + log_val)
    out_ref[...] = jnp.clip(bucket, 0, half - 1) + neg


def _sc_lookup_kernel(table_hbm, idx_hbm, out_hbm, table_v, idx_v, rows_v):
    wid = lax.axis_index("s") * _SC_CORES + lax.axis_index("c")
    base = wid * _IDX_PER_WORKER
    pltpu.sync_copy(table_hbm, table_v)
    pltpu.sync_copy(idx_hbm.at[pl.ds(base, _IDX_PER_WORKER)], idx_v)

    for g in range(_IDX_PER_WORKER // 16):
        idx16 = idx_v[pl.ds(g * 16, 16)]
        row_base = idx16 * _NUM_HEADS
        for h in range(_NUM_HEADS):
            vals = plsc.load_gather(table_v, [row_base + h])
            rows_v[h, pl.ds(g * 16, 16)] = vals
    pltpu.sync_copy(rows_v, out_hbm.at[:, pl.ds(base, _IDX_PER_WORKER)])


_BAND = 91
_PATCH = 384
_PWIN = 640


def _expand_kernel(shift_ref, diag_ref, out_ref, wb_ref):
    i = pl.program_id(0)
    shift = shift_ref[0]
    base_thr = i * _BQ - _BAND - shift
    lb = jnp.clip((base_thr >> 7) << 7, 0, _K - _PATCH)
    lb = pl.multiple_of(lb, 128)

    start = _Q - (i + 1) * _BQ
    pos = start + lb
    pos_al = pl.multiple_of(jnp.clip((pos >> 7) << 7, 0, _D - _PWIN), 128)
    awin = diag_ref[:, pl.ds(pos_al, _PWIN)]
    wb_ref[...] = pltpu.roll(awin, pos_al - pos + _PWIN, axis=1)

    c_lo = jnp.broadcast_to(diag_ref[:, 0:1], (_NUM_HEADS, _K))
    c_hi = jnp.broadcast_to(diag_ref[:, _D - 1:_D], (_NUM_HEADS, _K))
    lane = jax.lax.broadcasted_iota(jnp.int32, (_NUM_HEADS, _K), 1)
    fill = jnp.where(lane <= base_thr, c_lo, c_hi)
    out_ref[...] = jnp.broadcast_to(fill[:, None, :], (_NUM_HEADS, _BQ, _K))
    for r in range(_BQ):
        off = _BQ - 1 - r
        out_ref[:, r, pl.ds(lb, _PATCH)] = wb_ref[:, off:off + _PATCH]


def kernel(query_len, key_len, relative_attention_bias):
    shift = jnp.asarray(key_len - query_len, jnp.int32).reshape(1)

    buckets = pl.pallas_call(
        _bucket_kernel,
        grid_spec=pltpu.PrefetchScalarGridSpec(
            num_scalar_prefetch=1,
            grid=(1,),
            in_specs=[],
            out_specs=pl.BlockSpec((1, _D), lambda i, s: (0, 0)),
        ),
        out_shape=jax.ShapeDtypeStruct((1, _D), jnp.int32),
    )(shift).reshape(_D)

    lookup = pl.kernel(
        _sc_lookup_kernel,
        out_type=jax.ShapeDtypeStruct((_NUM_HEADS, _D), jnp.float32),
        mesh=plsc.VectorSubcoreMesh(core_axis_name="c", subcore_axis_name="s"),
        scratch_types=[
            pltpu.VMEM((_NUM_BUCKETS * _NUM_HEADS,), jnp.float32),
            pltpu.VMEM((_IDX_PER_WORKER,), jnp.int32),
            pltpu.VMEM((_NUM_HEADS, _IDX_PER_WORKER), jnp.float32),
        ],
        compiler_params=pltpu.CompilerParams(needs_layout_passes=False),
    )
    diag = lookup(relative_attention_bias.reshape(-1), buckets)

    out = pl.pallas_call(
        _expand_kernel,
        grid_spec=pltpu.PrefetchScalarGridSpec(
            num_scalar_prefetch=1,
            grid=(_Q // _BQ,),
            in_specs=[pl.BlockSpec((_NUM_HEADS, _D), lambda i, s: (0, 0))],
            out_specs=pl.BlockSpec((_NUM_HEADS, _BQ, _K), lambda i, s: (0, i, 0)),
            scratch_shapes=[pltpu.VMEM((_NUM_HEADS, _PWIN), jnp.float32)],
        ),
        out_shape=jax.ShapeDtypeStruct((_NUM_HEADS, _Q, _K), jnp.float32),
        compiler_params=pltpu.CompilerParams(
            dimension_semantics=("arbitrary",),
        ),
    )(shift, diag)
    return out[None]
